# baseline (speedup 1.0000x reference)
import jax
import jax.numpy as jnp
from jax import lax
from jax.experimental import pallas as pl
from jax.experimental.pallas import tpu as pltpu
from jax.experimental.pallas import tpu_sc as plsc

_S = 2048
_D = 16
_NW = 32
_RPW = _S // _NW
_UNROLL = 16


def _sc_body(x_hbm, w_hbm, out_hbm, x_v, w_v, ob0, ob1, sem0, sem1):
    c = lax.axis_index("c")
    s = lax.axis_index("s")
    wid = s * 2 + c
    base = wid * _RPW

    pltpu.sync_copy(x_hbm, x_v)
    pltpu.sync_copy(w_hbm.at[pl.ds(base, _RPW)], w_v)

    bufs = (ob0, ob1)
    sems = (sem0, sem1)

    def compute_krow(r, k, ob):
        wrow = w_v[r, :]
        wk = wrow[k]

        def inner(j0, carry):
            for u in range(_UNROLL):
                j = (j0 * _UNROLL + u) * 16
                ob[pl.ds(j, 16)] = x_v[pl.ds(j, 16)] + wk
            return carry

        lax.fori_loop(0, _S // (16 * _UNROLL), inner, 0)

    def row(r, carry):
        for k in range(_D):
            ob = bufs[k % 2]
            sem = sems[k % 2]
            dst = out_hbm.at[base + r, k]
            if k >= 2:
                pltpu.make_async_copy(ob, dst, sem).wait()
            else:
                @pl.when(r > 0)
                def _(ob=ob, dst=dst, sem=sem):
                    pltpu.make_async_copy(ob, dst, sem).wait()
            compute_krow(r, k, ob)
            pltpu.async_copy(ob, dst, sem)
        return carry

    lax.fori_loop(0, _RPW, row, 0)

    drain = out_hbm.at[base, 0]
    pltpu.make_async_copy(ob0, drain, sem0).wait()
    pltpu.make_async_copy(ob1, drain, sem1).wait()


def kernel(x, pos_embed_weight):
    seq_len, batch_size = x.shape
    _, dim = pos_embed_weight.shape

    xf = x.reshape(seq_len)
    w = pos_embed_weight[:seq_len]

    run = pl.kernel(
        _sc_body,
        out_type=jax.ShapeDtypeStruct((seq_len, dim, seq_len), jnp.float32),
        mesh=plsc.VectorSubcoreMesh(core_axis_name="c", subcore_axis_name="s"),
        scratch_types=[
            pltpu.VMEM((seq_len,), jnp.float32),
            pltpu.VMEM((_RPW, dim), jnp.float32),
            pltpu.VMEM((seq_len,), jnp.float32),
            pltpu.VMEM((seq_len,), jnp.float32),
            pltpu.SemaphoreType.DMA,
            pltpu.SemaphoreType.DMA,
        ],
        compiler_params=pltpu.CompilerParams(use_tc_tiling_on_sc=False),
    )
    out3 = run(xf, w)
    return jnp.transpose(out3, (0, 2, 1))


# speedup vs baseline: 4.7740x; 4.7740x over previous
import jax
import jax.numpy as jnp
from jax.experimental import pallas as pl


def _bcast_add_kernel(w_ref, xt_ref, o_ref):
    w = w_ref[...]
    xt = xt_ref[...]
    o_ref[...] = w[:, :, None] + xt[None, :, :]


def kernel(x, pos_embed_weight):
    seq_len, batch_size = x.shape
    _, dim = pos_embed_weight.shape

    w = pos_embed_weight[:seq_len]
    xt = x.reshape(1, seq_len)

    BI = 32
    out3 = pl.pallas_call(
        _bcast_add_kernel,
        grid=(seq_len // BI,),
        in_specs=[
            pl.BlockSpec((BI, dim), lambda i: (i, 0)),
            pl.BlockSpec((1, seq_len), lambda i: (0, 0)),
        ],
        out_specs=pl.BlockSpec((BI, dim, seq_len), lambda i: (i, 0, 0)),
        out_shape=jax.ShapeDtypeStruct((seq_len, dim, seq_len), jnp.float32),
    )(w, xt)

    return jnp.transpose(out3, (0, 2, 1))
